# Initial kernel scaffold; baseline (speedup 1.0000x reference)
import jax
import jax.numpy as jnp
from jax.experimental import pallas as pl


def kernel(features, unit_means):
    raise NotImplementedError("write your pallas kernel here")


# speedup vs baseline: 34.4608x; 34.4608x over previous
import functools

import jax
import jax.numpy as jnp
from jax.experimental import pallas as pl

_TOPK = 5
_BLOCK_N = 256


def _topk_kernel(f_ref, mt_ref, s_ref, i_ref):
    f = f_ref[...]
    mt = mt_ref[...]
    x2 = jnp.sum(f * f, axis=1, keepdims=True)
    m2 = jnp.sum(mt * mt, axis=0, keepdims=True)
    dot = jnp.dot(f, mt, preferred_element_type=jnp.float32)
    scores = -0.5 * (x2 - 2.0 * dot + m2)

    b, k = scores.shape
    iota = jax.lax.broadcasted_iota(jnp.int32, (b, k), 1)
    work = scores
    top_s = []
    top_i = []
    for _ in range(_TOPK):
        cur_max = jnp.max(work, axis=1, keepdims=True)
        cur_idx = jnp.min(jnp.where(work == cur_max, iota, k),
                          axis=1, keepdims=True)
        top_s.append(cur_max)
        top_i.append(cur_idx)
        work = jnp.where(iota == cur_idx, -jnp.inf, work)
    s_ref[...] = jnp.concatenate(top_s, axis=1)
    i_ref[...] = jnp.concatenate(top_i, axis=1)


@functools.partial(jax.jit, static_argnames=())
def kernel(features, unit_means):
    n, d = features.shape
    k = unit_means.shape[0]
    mt = unit_means.T
    grid = (n // _BLOCK_N,)
    out_s, out_i = pl.pallas_call(
        _topk_kernel,
        grid=grid,
        in_specs=[
            pl.BlockSpec((_BLOCK_N, d), lambda i: (i, 0)),
            pl.BlockSpec((d, k), lambda i: (0, 0)),
        ],
        out_specs=[
            pl.BlockSpec((_BLOCK_N, _TOPK), lambda i: (i, 0)),
            pl.BlockSpec((_BLOCK_N, _TOPK), lambda i: (i, 0)),
        ],
        out_shape=[
            jax.ShapeDtypeStruct((n, _TOPK), jnp.float32),
            jax.ShapeDtypeStruct((n, _TOPK), jnp.int32),
        ],
    )(features, mt)
    return out_s, out_i


# speedup vs baseline: 35.1478x; 1.0199x over previous
import functools

import jax
import jax.numpy as jnp
from jax.experimental import pallas as pl

_TOPK = 5
_BLOCK_N = 256


def _topk_kernel(f_ref, mt_ref, s_ref, i_ref):
    f = f_ref[...]
    mt = mt_ref[...]
    mh = -0.5 * jnp.sum(mt * mt, axis=0, keepdims=True)
    dot = jnp.dot(f, mt, preferred_element_type=jnp.float32)
    g = dot + mh

    b, k = g.shape
    iota = jax.lax.broadcasted_iota(jnp.int32, (b, k), 1)
    work = g
    top_s = []
    top_i = []
    for _ in range(_TOPK):
        cur_max = jnp.max(work, axis=1, keepdims=True)
        hit = work == cur_max
        cur_idx = jnp.min(jnp.where(hit, iota, k),
                          axis=1, keepdims=True)
        top_s.append(cur_max)
        top_i.append(cur_idx)
        work = jnp.where(iota == cur_idx, -jnp.inf, work)
    xh = -0.5 * jnp.sum(f * f, axis=1, keepdims=True)
    s_ref[...] = jnp.concatenate(top_s, axis=1) + xh
    i_ref[...] = jnp.concatenate(top_i, axis=1)


@functools.partial(jax.jit, static_argnames=())
def kernel(features, unit_means):
    n, d = features.shape
    k = unit_means.shape[0]
    mt = unit_means.T
    grid = (n // _BLOCK_N,)
    out_s, out_i = pl.pallas_call(
        _topk_kernel,
        grid=grid,
        in_specs=[
            pl.BlockSpec((_BLOCK_N, d), lambda i: (i, 0)),
            pl.BlockSpec((d, k), lambda i: (0, 0)),
        ],
        out_specs=[
            pl.BlockSpec((_BLOCK_N, _TOPK), lambda i: (i, 0)),
            pl.BlockSpec((_BLOCK_N, _TOPK), lambda i: (i, 0)),
        ],
        out_shape=[
            jax.ShapeDtypeStruct((n, _TOPK), jnp.float32),
            jax.ShapeDtypeStruct((n, _TOPK), jnp.int32),
        ],
    )(features, mt)
    return out_s, out_i


# speedup vs baseline: 53.2746x; 1.5157x over previous
import functools

import jax
import jax.numpy as jnp
from jax.experimental import pallas as pl

_TOPK = 5
_BLOCK_N = 256


def _topk_kernel(f_ref, mt_ref, s_ref, i_ref):
    f = f_ref[...]
    mt = mt_ref[...]
    mh = -0.5 * jnp.sum(mt * mt, axis=0, keepdims=True)
    dot = jnp.dot(f, mt, preferred_element_type=jnp.float32)
    g = dot + mh

    b, k = g.shape
    iota_f = jax.lax.broadcasted_iota(jnp.int32, (b, k), 1).astype(jnp.float32)
    work = g
    top_s = []
    top_i = []
    for _ in range(_TOPK):
        cur_max = jnp.max(work, axis=1, keepdims=True)
        hit = work == cur_max
        cur_idx = jnp.min(jnp.where(hit, iota_f, float(k)),
                          axis=1, keepdims=True)
        top_s.append(cur_max)
        top_i.append(cur_idx)
        work = jnp.where(hit, -jnp.inf, work)
    xh = -0.5 * jnp.sum(f * f, axis=1, keepdims=True)
    s_ref[...] = jnp.concatenate(top_s, axis=1) + xh
    i_ref[...] = jnp.concatenate(top_i, axis=1).astype(jnp.int32)


@functools.partial(jax.jit, static_argnames=())
def kernel(features, unit_means):
    n, d = features.shape
    k = unit_means.shape[0]
    mt = unit_means.T
    grid = (n // _BLOCK_N,)
    out_s, out_i = pl.pallas_call(
        _topk_kernel,
        grid=grid,
        in_specs=[
            pl.BlockSpec((_BLOCK_N, d), lambda i: (i, 0)),
            pl.BlockSpec((d, k), lambda i: (0, 0)),
        ],
        out_specs=[
            pl.BlockSpec((_BLOCK_N, _TOPK), lambda i: (i, 0)),
            pl.BlockSpec((_BLOCK_N, _TOPK), lambda i: (i, 0)),
        ],
        out_shape=[
            jax.ShapeDtypeStruct((n, _TOPK), jnp.float32),
            jax.ShapeDtypeStruct((n, _TOPK), jnp.int32),
        ],
    )(features, mt)
    return out_s, out_i
